# Initial kernel scaffold; baseline (speedup 1.0000x reference)
import jax
import jax.numpy as jnp
from jax.experimental import pallas as pl


def kernel(hidden, tokens):
    raise NotImplementedError("write your pallas kernel here")


# speedup vs baseline: 1.0759x; 1.0759x over previous
import functools

import jax
import jax.numpy as jnp
from jax.experimental import pallas as pl
from jax.experimental.pallas import tpu as pltpu

N = 8192
DIM = 4096
K = 3
ROWS_PER_BLOCK = 1024
GRID = N // ROWS_PER_BLOCK
SUBL = ROWS_PER_BLOCK // 128


def _norms_body(h_ref, out_ref):
    x = h_ref[...]
    s = jnp.sum(x * x, axis=1)
    out_ref[...] = s.reshape(SUBL, 128)


def _select_body(norms_ref, tokens_ref, hid_ref, anchors_ref, meta_ref, sem):
    v = norms_ref[...]
    row = jax.lax.broadcasted_iota(jnp.int32, v.shape, 0)
    lane = jax.lax.broadcasted_iota(jnp.int32, v.shape, 1)
    gidx = row * 128 + lane
    big = jnp.int32(2**31 - 1)

    idxs = []
    for _ in range(K):
        m = jnp.max(v)
        cand = jnp.where(v == m, gidx, big)
        ik = jnp.min(cand)
        idxs.append(ik)
        v = jnp.where(gidx == ik, jnp.float32(-1.0), v)

    t = tokens_ref[...]
    toks = [jnp.sum(jnp.where(gidx == ik, t, 0)) for ik in idxs]

    lane8 = jax.lax.broadcasted_iota(jnp.int32, (8, 128), 1)
    meta = jnp.where(lane8 == 0, toks[0],
                     jnp.where(lane8 == 1, toks[1],
                               jnp.where(lane8 == 2, toks[2], 0)))
    meta_ref[...] = meta

    for k, ik in enumerate(idxs):
        cp = pltpu.make_async_copy(hid_ref.at[pl.ds(ik, 1), :],
                                   anchors_ref.at[pl.ds(k, 1), :], sem)
        cp.start()
        cp.wait()


@jax.jit
def _run(hidden, tokens_2d):
    norms2 = pl.pallas_call(
        _norms_body,
        grid=(GRID,),
        in_specs=[pl.BlockSpec((ROWS_PER_BLOCK, DIM), lambda i: (i, 0))],
        out_specs=pl.BlockSpec((SUBL, 128), lambda i: (i, 0)),
        out_shape=jax.ShapeDtypeStruct((N // 128, 128), jnp.float32),
    )(hidden)

    anchors, meta = pl.pallas_call(
        _select_body,
        in_specs=[
            pl.BlockSpec(memory_space=pltpu.VMEM),
            pl.BlockSpec(memory_space=pltpu.VMEM),
            pl.BlockSpec(memory_space=pl.ANY),
        ],
        out_specs=[
            pl.BlockSpec(memory_space=pltpu.VMEM),
            pl.BlockSpec(memory_space=pltpu.VMEM),
        ],
        out_shape=[
            jax.ShapeDtypeStruct((K, DIM), jnp.float32),
            jax.ShapeDtypeStruct((8, 128), jnp.int32),
        ],
        scratch_shapes=[pltpu.SemaphoreType.DMA],
    )(norms2, tokens_2d, hidden)
    return anchors, meta


def kernel(hidden, tokens):
    tokens_2d = tokens.astype(jnp.int32).reshape(N // 128, 128)
    anchors, meta = _run(hidden, tokens_2d)
    sel_tokens = meta[0, :K].astype(tokens.dtype)
    return anchors, sel_tokens
